# baseline (speedup 1.0000x reference)
import functools

import jax
import jax.numpy as jnp
import numpy as np
from jax import lax
from jax.experimental import pallas as pl
from jax.experimental.pallas import tpu as pltpu
from jax.experimental.pallas import tpu_sc as plsc

N = 10000
E = 320000
SH = 9
FC = 100
H = 32
MAXR = 5.0
INV_SQRT_NEIGH = 1.0 / np.sqrt(32.0)

NC = 2
NS = 16
NW = NC * NS
NPAD = 112
N16 = N + NPAD
EPW = 80 * 128
EPAD = NW * EPW
C16 = EPW // 16
C128 = EPW // 128
RPS = N16 // NS

TCB = 2048
NB_E = EPAD // TCB
TCN = 1024
NB_N = (N16 + TCN - 1) // TCN

f32 = jnp.float32
i32 = jnp.int32


@functools.lru_cache(maxsize=None)
def _mesh():
    return plsc.VectorSubcoreMesh(
        core_axis_name="c", subcore_axis_name="s",
        num_cores=NC, num_subcores=NS)


def _sc_edge_vec_body(posx, posy, posz, srcr, dstr, ev, px, py, pz,
                      sbuf, dbuf, ex, ey, ez):
    wid = lax.axis_index("c") * NS + lax.axis_index("s")
    pltpu.sync_copy(posx, px)
    pltpu.sync_copy(posy, py)
    pltpu.sync_copy(posz, pz)
    pltpu.sync_copy(srcr.at[wid], sbuf)
    pltpu.sync_copy(dstr.at[wid], dbuf)

    def body(i, carry):
        o = i * 16
        s = sbuf[pl.ds(o, 16)]
        t = dbuf[pl.ds(o, 16)]
        ex[pl.ds(o, 16)] = plsc.load_gather(px, [s]) - plsc.load_gather(px, [t])
        ey[pl.ds(o, 16)] = plsc.load_gather(py, [s]) - plsc.load_gather(py, [t])
        ez[pl.ds(o, 16)] = plsc.load_gather(pz, [s]) - plsc.load_gather(pz, [t])
        return carry

    lax.fori_loop(0, C16, body, 0)
    ebase = pl.multiple_of(wid * EPW, 128)
    pltpu.sync_copy(ex, ev.at[0, pl.ds(ebase, EPW)])
    pltpu.sync_copy(ey, ev.at[1, pl.ds(ebase, EPW)])
    pltpu.sync_copy(ez, ev.at[2, pl.ds(ebase, EPW)])


@functools.lru_cache(maxsize=None)
def _sc_edge_vec_kernel():
    return pl.kernel(
        _sc_edge_vec_body,
        out_type=jax.ShapeDtypeStruct((3, EPAD), f32),
        mesh=_mesh(),
        compiler_params=pltpu.CompilerParams(needs_layout_passes=False, use_tc_tiling_on_sc=False),
        scratch_types=[
            pltpu.VMEM((N16,), f32),
            pltpu.VMEM((N16,), f32),
            pltpu.VMEM((N16,), f32),
            pltpu.VMEM((EPW,), i32),
            pltpu.VMEM((EPW,), i32),
            pltpu.VMEM((EPW,), f32),
            pltpu.VMEM((EPW,), f32),
            pltpu.VMEM((EPW,), f32),
        ],
    )


def _sc_scatter2_body(callr, dstr, srcr, p0, s2, acc0, acc2, dstb, srcb,
                      mbuf0, cbuf0, mbuf1, cbuf1, zbuf, sem0, sem1):
    c = lax.axis_index("c")
    s = lax.axis_index("s")
    wid = c * NS + s
    z16 = jnp.zeros((16,), f32)

    def zbody(j, carry):
        zbuf[j, pl.ds(0, 16)] = z16
        zbuf[j, pl.ds(16, 16)] = z16
        return carry

    lax.fori_loop(0, RPS, zbody, 0)
    pltpu.sync_copy(zbuf, acc0.at[pl.ds(s * RPS, RPS)])
    pltpu.sync_copy(zbuf, acc2.at[pl.ds(s * RPS, RPS)])
    crow = pl.multiple_of(wid * C128, 8)
    pltpu.sync_copy(dstr.at[pl.ds(crow, C128)], dstb)
    pltpu.sync_copy(srcr.at[pl.ds(crow, C128)], srcb)

    def start(j, mb, cb, sem):
        base = pl.multiple_of(wid * EPW + j * 128, 128)
        pltpu.async_copy(callr.at[pl.ds(base, 128), pl.ds(0, H)], mb, sem)
        pltpu.async_copy(callr.at[pl.ds(base, 128), pl.ds(2 * H, H)], cb, sem)

    def wait(mb, cb, sem):
        pltpu.make_async_copy(callr.at[pl.ds(0, 128), pl.ds(0, H)], mb,
                              sem).wait()
        pltpu.make_async_copy(callr.at[pl.ds(0, 128), pl.ds(0, H)], cb,
                              sem).wait()

    start(0, mbuf0, cbuf0, sem0)
    plsc.subcore_barrier()

    def body(jo, carry):
        a = 2 * jo
        start(a + 1, mbuf1, cbuf1, sem1)
        wait(mbuf0, cbuf0, sem0)
        pltpu.sync_copy(mbuf0, acc0.at[dstb.at[a]], add=True)
        pltpu.sync_copy(cbuf0, acc2.at[srcb.at[a]], add=True)

        @pl.when(jo < C128 // 2 - 1)
        def _():
            start(a + 2, mbuf0, cbuf0, sem0)

        wait(mbuf1, cbuf1, sem1)
        pltpu.sync_copy(mbuf1, acc0.at[dstb.at[a + 1]], add=True)
        pltpu.sync_copy(cbuf1, acc2.at[srcb.at[a + 1]], add=True)
        return carry

    lax.fori_loop(0, C128 // 2, body, 0)
    plsc.subcore_barrier()
    pltpu.sync_copy(acc0.at[pl.ds(s * RPS, RPS)], zbuf)
    pltpu.sync_copy(zbuf, p0.at[c, pl.ds(s * RPS, RPS)])
    pltpu.sync_copy(acc2.at[pl.ds(s * RPS, RPS)], zbuf)
    pltpu.sync_copy(zbuf, s2.at[c, pl.ds(s * RPS, RPS)])


@functools.lru_cache(maxsize=None)
def _sc_scatter2_kernel():
    return pl.kernel(
        _sc_scatter2_body,
        out_type=(
            jax.ShapeDtypeStruct((NC, N16, H), f32),
            jax.ShapeDtypeStruct((NC, N16, H), f32),
        ),
        mesh=_mesh(),
        compiler_params=pltpu.CompilerParams(needs_layout_passes=False, use_tc_tiling_on_sc=False),
        scratch_types=[
            pltpu.VMEM_SHARED((N16, H), f32),
            pltpu.VMEM_SHARED((N16, H), f32),
            pltpu.VMEM((C128, 128), i32),
            pltpu.VMEM((C128, 128), i32),
            pltpu.VMEM((128, H), f32),
            pltpu.VMEM((128, H), f32),
            pltpu.VMEM((128, H), f32),
            pltpu.VMEM((128, H), f32),
            pltpu.VMEM((RPS, H), f32),
            pltpu.SemaphoreType.DMA,
            pltpu.SemaphoreType.DMA,
        ],
    )


def _sc_gather_mul_scatter_body(y1, callr, srcr, dstr, p1, acc, srcb,
                                dstb, rows0, cbuf0, rows1, cbuf1, zbuf,
                                sem0, sem1):
    c = lax.axis_index("c")
    s = lax.axis_index("s")
    wid = c * NS + s
    z16 = jnp.zeros((16,), f32)
    nrow = pl.multiple_of(s * RPS, 8)

    def zbody(j, carry):
        zbuf[j, pl.ds(0, 16)] = z16
        zbuf[j, pl.ds(16, 16)] = z16
        return carry

    lax.fori_loop(0, RPS, zbody, 0)
    pltpu.sync_copy(zbuf, acc.at[pl.ds(nrow, RPS)])
    crow = pl.multiple_of(wid * C128, 8)
    pltpu.sync_copy(srcr.at[pl.ds(crow, C128)], srcb)
    pltpu.sync_copy(dstr.at[pl.ds(crow, C128)], dstb)

    def start(j, rb, cb, sem):
        base = pl.multiple_of(wid * EPW + j * 128, 128)
        pltpu.async_copy(y1.at[srcb.at[j]], rb, sem)
        pltpu.async_copy(callr.at[pl.ds(base, 128), pl.ds(H, H)], cb, sem)

    def wait_mul(rb, cb, sem):
        pltpu.make_async_copy(callr.at[pl.ds(0, 128), pl.ds(0, H)], rb,
                              sem).wait()
        pltpu.make_async_copy(callr.at[pl.ds(0, 128), pl.ds(0, H)], cb,
                              sem).wait()

        def mbody(r, carry):
            rb[r, pl.ds(0, 16)] = rb[r, pl.ds(0, 16)] * cb[r, pl.ds(0, 16)]
            rb[r, pl.ds(16, 16)] = rb[r, pl.ds(16, 16)] * cb[r, pl.ds(16, 16)]
            return carry

        lax.fori_loop(0, 128, mbody, 0)

    plsc.subcore_barrier()
    start(0, rows0, cbuf0, sem0)

    def body(jo, carry):
        a = 2 * jo
        start(a + 1, rows1, cbuf1, sem1)
        wait_mul(rows0, cbuf0, sem0)
        pltpu.sync_copy(rows0, acc.at[dstb.at[a]], add=True)

        @pl.when(jo < C128 // 2 - 1)
        def _():
            start(a + 2, rows0, cbuf0, sem0)

        wait_mul(rows1, cbuf1, sem1)
        pltpu.sync_copy(rows1, acc.at[dstb.at[a + 1]], add=True)
        return carry

    lax.fori_loop(0, C128 // 2, body, 0)
    plsc.subcore_barrier()
    pltpu.sync_copy(acc.at[pl.ds(s * RPS, RPS)], zbuf)
    pltpu.sync_copy(zbuf, p1.at[c, pl.ds(s * RPS, RPS)])


@functools.lru_cache(maxsize=None)
def _sc_gather_mul_scatter_kernel():
    return pl.kernel(
        _sc_gather_mul_scatter_body,
        out_type=jax.ShapeDtypeStruct((NC, N16, H), f32),
        mesh=_mesh(),
        compiler_params=pltpu.CompilerParams(needs_layout_passes=False, use_tc_tiling_on_sc=False),
        scratch_types=[
            pltpu.VMEM_SHARED((N16, H), f32),
            pltpu.VMEM((C128, 128), i32),
            pltpu.VMEM((C128, 128), i32),
            pltpu.VMEM((128, H), f32),
            pltpu.VMEM((128, H), f32),
            pltpu.VMEM((128, H), f32),
            pltpu.VMEM((128, H), f32),
            pltpu.VMEM((RPS, H), f32),
            pltpu.SemaphoreType.DMA,
            pltpu.SemaphoreType.DMA,
        ],
    )


def _tc_edge_body(ev_ref, wf1_ref, bf1_ref, wf2_ref, wsh_ref, ws0_ref,
                  call_ref):
    x = ev_ref[0:1, :]
    y = ev_ref[1:2, :]
    z = ev_ref[2:3, :]
    d = jnp.sqrt(x * x + y * y + z * z)
    rinv = 1.0 / (d + 1e-9)
    nx = x * rinv
    ny = y * rinv
    nz = z * rinv
    s3 = np.sqrt(3.0).astype(np.float32)
    s15 = np.sqrt(15.0).astype(np.float32)
    s5 = np.sqrt(5.0).astype(np.float32)
    u = jnp.clip(d * (1.0 / MAXR), 0.0, 1.0)
    xx = np.float32(0.5 * np.pi) * u
    x2 = xx * xx
    emb = 1.0 + x2 * (np.float32(-1 / 2) + x2 * (np.float32(1 / 24) + x2 * (
        np.float32(-1 / 720) + x2 * (np.float32(1 / 40320)
                                     + x2 * np.float32(-1 / 3628800)))))
    geomT = jnp.concatenate([
        jnp.ones_like(nx), s3 * nx, s3 * ny, s3 * nz,
        s15 * nx * ny, s15 * ny * nz, 0.5 * s5 * (3.0 * nz * nz - 1.0),
        s15 * nx * nz, 0.5 * s15 * (nx * nx - ny * ny),
        emb], axis=0)
    geom = geomT.T
    sh = geom[:, 0:SH].astype(jnp.bfloat16)
    embc = geom[:, SH:SH + 1]
    for i in range(3):
        hid = jnp.maximum(embc * wf1_ref[i] + bf1_ref[i],
                          0.0).astype(jnp.bfloat16)
        w = jnp.dot(hid, wf2_ref[i], preferred_element_type=f32)
        csh = jnp.dot(sh, wsh_ref[i], preferred_element_type=f32)
        cc = csh * w * INV_SQRT_NEIGH
        if i == 0:
            cc = cc * ws0_ref[...]
        call_ref[:, pl.ds(i * H, H)] = cc


def _tc_edge_feats(ev, wf1, bf1, wf2, wsh, ws0):
    full = lambda shape: pl.BlockSpec(shape, lambda b: (0,) * len(shape))
    return pl.pallas_call(
        _tc_edge_body,
        grid=(NB_E,),
        in_specs=[
            pl.BlockSpec((3, TCB), lambda b: (0, b)),
            full((3, 1, FC)),
            full((3, 1, FC)),
            full((3, FC, H)),
            full((3, SH, H)),
            full((1, H)),
        ],
        out_specs=pl.BlockSpec((TCB, 128), lambda b: (b, 0)),
        out_shape=jax.ShapeDtypeStruct((EPAD, 128), f32),
    )(ev, wf1, bf1, wf2, wsh, ws0)


def _tc_layer1_body(p0_ref, na_ref, wo0a_ref, wo0b_ref, wo0c_ref, ws1_ref,
                    x1_ref, y1_ref):
    agg = p0_ref[0] + p0_ref[1]
    x1 = jnp.tanh(jnp.dot(agg, wo0a_ref[...]) + wo0b_ref[...]
                  + na_ref[...] * wo0c_ref[...])
    x1_ref[...] = x1
    y1_ref[...] = jnp.dot(x1, ws1_ref[...])


def _tc_layer1(p0, na, wo0a, wo0b, wo0c, ws1):
    full = lambda shape: pl.BlockSpec(shape, lambda b: (0,) * len(shape))
    return pl.pallas_call(
        _tc_layer1_body,
        grid=(NB_N,),
        in_specs=[
            pl.BlockSpec((NC, TCN, H), lambda b: (0, b, 0)),
            pl.BlockSpec((TCN, 1), lambda b: (b, 0)),
            full((H, 64)),
            full((1, 64)),
            full((1, 64)),
            full((64, H)),
        ],
        out_specs=[
            pl.BlockSpec((TCN, 64), lambda b: (b, 0)),
            pl.BlockSpec((TCN, H), lambda b: (b, 0)),
        ],
        out_shape=[
            jax.ShapeDtypeStruct((N16, 64), f32),
            jax.ShapeDtypeStruct((N16, H), f32),
        ],
    )(p0, na, wo0a, wo0b, wo0c, ws1)


def _tc_final_body(p1_ref, s2_ref, x1_ref, na_ref, wo1a_ref, wo1b_ref,
                   wo1c_ref, ws2_ref, wo2a_ref, wo2b_ref, wo2c_ref,
                   t_ref, sx2_ref, sna_ref, out_ref):
    b = pl.program_id(0)
    agg1 = p1_ref[0] + p1_ref[1]
    x2 = jnp.tanh(jnp.dot(agg1, wo1a_ref[...])
                  + jnp.dot(x1_ref[...], wo1b_ref[...])
                  + na_ref[...] * wo1c_ref[...])
    rows = lax.broadcasted_iota(i32, (TCN, 1), 0) + b * TCN
    mask = rows < N
    x2m = jnp.where(mask, x2, 0.0)
    sb = jnp.where(mask, s2_ref[0] + s2_ref[1], 0.0)
    tb = lax.dot_general(x2m, sb, (((0,), (0,)), ((), ())))
    sx2b = jnp.sum(x2m, axis=0, keepdims=True)
    snab = jnp.sum(jnp.where(mask, na_ref[...], 0.0), axis=0,
                   keepdims=True)

    @pl.when(b == 0)
    def _():
        t_ref[...] = tb
        sx2_ref[...] = sx2b
        sna_ref[...] = snab

    @pl.when(b > 0)
    def _():
        t_ref[...] = t_ref[...] + tb
        sx2_ref[...] = sx2_ref[...] + sx2b
        sna_ref[...] = sna_ref[...] + snab

    @pl.when(b == NB_N - 1)
    def _():
        sumagg2 = jnp.sum(ws2_ref[...] * t_ref[...], axis=0,
                          keepdims=True)
        out_ref[...] = (jnp.dot(sumagg2, wo2a_ref[...])
                        + jnp.dot(sx2_ref[...], wo2b_ref[...])
                        + sna_ref[...] * wo2c_ref[...]) * np.float32(
                            1.0 / np.sqrt(float(N)))


def _tc_final(p1, s2, x1, na, wo1a, wo1b, wo1c, ws2, wo2a, wo2b, wo2c):
    full = lambda shape: pl.BlockSpec(shape, lambda b: (0,) * len(shape))
    outs = pl.pallas_call(
        _tc_final_body,
        grid=(NB_N,),
        in_specs=[
            pl.BlockSpec((NC, TCN, H), lambda b: (0, b, 0)),
            pl.BlockSpec((NC, TCN, H), lambda b: (0, b, 0)),
            pl.BlockSpec((TCN, 64), lambda b: (b, 0)),
            pl.BlockSpec((TCN, 1), lambda b: (b, 0)),
            full((H, 64)),
            full((64, 64)),
            full((1, 64)),
            full((64, H)),
            full((H, 1)),
            full((64, 1)),
            full((1, 1)),
        ],
        out_specs=[full((64, H)), full((1, 64)), full((1, 1)), full((1, 1))],
        out_shape=[
            jax.ShapeDtypeStruct((64, H), f32),
            jax.ShapeDtypeStruct((1, 64), f32),
            jax.ShapeDtypeStruct((1, 1), f32),
            jax.ShapeDtypeStruct((1, 1), f32),
        ],
    )(p1, s2, x1, na, wo1a, wo1b, wo1c, ws2, wo2a, wo2b, wo2c)
    return outs[3]


def kernel(pos, node_attr, edge_index, Wf1_0, bf1_0, Wf2_0, Ws_0, Wsh_0, Wo_0,
           Wf1_1, bf1_1, Wf2_1, Ws_1, Wsh_1, Wo_1, Wf1_2, bf1_2, Wf2_2, Ws_2,
           Wsh_2, Wo_2):
    pos = pos.astype(f32)
    node_attr = node_attr.astype(f32)
    src = edge_index[0].astype(i32)
    dst = edge_index[1].astype(i32)
    padidx = jnp.full((EPAD - E,), N, i32)
    src_p = jnp.concatenate([src, padidx])
    dst_p = jnp.concatenate([dst, padidx])
    pos_p = jnp.concatenate([pos, jnp.zeros((NPAD, 3), f32)], axis=0)
    na_p = jnp.concatenate([node_attr, jnp.zeros((NPAD, 1), f32)], axis=0)

    wf1 = jnp.stack([Wf1_0.reshape(1, FC), Wf1_1.reshape(1, FC),
                     Wf1_2.reshape(1, FC)])
    bf1 = jnp.stack([bf1_0.reshape(1, FC), bf1_1.reshape(1, FC),
                     bf1_2.reshape(1, FC)])
    wf2 = jnp.stack([Wf2_0, Wf2_1, Wf2_2]).astype(jnp.bfloat16)
    wsh = jnp.stack([Wsh_0, Wsh_1, Wsh_2]).astype(jnp.bfloat16)
    ws0 = Ws_0[0:1]

    ev = _sc_edge_vec_kernel()(pos_p[:, 0], pos_p[:, 1], pos_p[:, 2],
                               src_p.reshape(NW, EPW), dst_p.reshape(NW, EPW))
    call = _tc_edge_feats(ev, wf1, bf1, wf2, wsh, ws0)

    src_r = src_p.reshape(NW * C128, 128)
    dst_r = dst_p.reshape(NW * C128, 128)
    p0, s2 = _sc_scatter2_kernel()(call, dst_r, src_r)

    x1, y1 = _tc_layer1(p0, na_p, Wo_0[:H], Wo_0[H:H + 1], Wo_0[H + 18:H + 19],
                        Ws_1)

    p1 = _sc_gather_mul_scatter_kernel()(y1, call, src_r, dst_r)

    return _tc_final(p1, s2, x1, na_p, Wo_1[:H], Wo_1[H:H + 64],
                     Wo_1[H + 64:H + 65], Ws_2, Wo_2[:H], Wo_2[H:H + 64],
                     Wo_2[H + 64:H + 65])


# speedup vs baseline: 1.1452x; 1.1452x over previous
import functools

import jax
import jax.numpy as jnp
import numpy as np
from jax import lax
from jax.experimental import pallas as pl
from jax.experimental.pallas import tpu as pltpu
from jax.experimental.pallas import tpu_sc as plsc

N = 10000
E = 320000
SH = 9
FC = 100
H = 32
MAXR = 5.0
INV_SQRT_NEIGH = 1.0 / np.sqrt(32.0)

NC = 2
NS = 16
NW = NC * NS
NPAD = 112
N16 = N + NPAD
EPW = 80 * 128
EPAD = NW * EPW
C16 = EPW // 16
C128 = EPW // 128
RPS = N16 // NS

TCB = 2048
NB_E = EPAD // TCB
TCN = 1024
NB_N = (N16 + TCN - 1) // TCN

f32 = jnp.float32
i32 = jnp.int32


@functools.lru_cache(maxsize=None)
def _mesh():
    return plsc.VectorSubcoreMesh(
        core_axis_name="c", subcore_axis_name="s",
        num_cores=NC, num_subcores=NS)


def _sc_edge_vec_body(posx, posy, posz, srcr, dstr, ev, px, py, pz,
                      sbuf, dbuf, ex, ey, ez):
    wid = lax.axis_index("c") * NS + lax.axis_index("s")
    pltpu.sync_copy(posx, px)
    pltpu.sync_copy(posy, py)
    pltpu.sync_copy(posz, pz)
    pltpu.sync_copy(srcr.at[wid], sbuf)
    pltpu.sync_copy(dstr.at[wid], dbuf)

    def body(i, carry):
        o = i * 16
        s = sbuf[pl.ds(o, 16)]
        t = dbuf[pl.ds(o, 16)]
        ex[pl.ds(o, 16)] = plsc.load_gather(px, [s]) - plsc.load_gather(px, [t])
        ey[pl.ds(o, 16)] = plsc.load_gather(py, [s]) - plsc.load_gather(py, [t])
        ez[pl.ds(o, 16)] = plsc.load_gather(pz, [s]) - plsc.load_gather(pz, [t])
        return carry

    lax.fori_loop(0, C16, body, 0)
    ebase = pl.multiple_of(wid * EPW, 128)
    pltpu.sync_copy(ex, ev.at[0, pl.ds(ebase, EPW)])
    pltpu.sync_copy(ey, ev.at[1, pl.ds(ebase, EPW)])
    pltpu.sync_copy(ez, ev.at[2, pl.ds(ebase, EPW)])


@functools.lru_cache(maxsize=None)
def _sc_edge_vec_kernel():
    return pl.kernel(
        _sc_edge_vec_body,
        out_type=jax.ShapeDtypeStruct((3, EPAD), f32),
        mesh=_mesh(),
        compiler_params=pltpu.CompilerParams(needs_layout_passes=False, use_tc_tiling_on_sc=False),
        scratch_types=[
            pltpu.VMEM((N16,), f32),
            pltpu.VMEM((N16,), f32),
            pltpu.VMEM((N16,), f32),
            pltpu.VMEM((EPW,), i32),
            pltpu.VMEM((EPW,), i32),
            pltpu.VMEM((EPW,), f32),
            pltpu.VMEM((EPW,), f32),
            pltpu.VMEM((EPW,), f32),
        ],
    )


def _sc_scatter2_body(callr, dstr, srcr, p0, s2, acc0, acc2, dstb, srcb,
                      mbuf0, cbuf0, mbuf1, cbuf1, zbuf, sem0, sem1):
    c = lax.axis_index("c")
    s = lax.axis_index("s")
    wid = c * NS + s
    z16 = jnp.zeros((16,), f32)

    def zbody(j, carry):
        zbuf[j, pl.ds(0, 16)] = z16
        zbuf[j, pl.ds(16, 16)] = z16
        return carry

    lax.fori_loop(0, RPS, zbody, 0)
    pltpu.sync_copy(zbuf, acc0.at[pl.ds(s * RPS, RPS)])
    pltpu.sync_copy(zbuf, acc2.at[pl.ds(s * RPS, RPS)])
    crow = pl.multiple_of(wid * C128, 8)
    pltpu.sync_copy(dstr.at[pl.ds(crow, C128)], dstb)
    pltpu.sync_copy(srcr.at[pl.ds(crow, C128)], srcb)

    def start(j, mb, cb, sem):
        base = pl.multiple_of(wid * EPW + j * 128, 128)
        pltpu.async_copy(callr.at[pl.ds(base, 128), pl.ds(0, H)], mb, sem)
        pltpu.async_copy(callr.at[pl.ds(base, 128), pl.ds(2 * H, H)], cb, sem)

    def wait(mb, cb, sem):
        pltpu.make_async_copy(callr.at[pl.ds(0, 128), pl.ds(0, H)], mb,
                              sem).wait()
        pltpu.make_async_copy(callr.at[pl.ds(0, 128), pl.ds(0, H)], cb,
                              sem).wait()

    start(0, mbuf0, cbuf0, sem0)
    plsc.subcore_barrier()

    def body(jo, carry):
        a = 2 * jo
        start(a + 1, mbuf1, cbuf1, sem1)
        wait(mbuf0, cbuf0, sem0)
        pltpu.sync_copy(mbuf0, acc0.at[dstb.at[a]], add=True)
        pltpu.sync_copy(cbuf0, acc2.at[srcb.at[a]], add=True)

        @pl.when(jo < C128 // 2 - 1)
        def _():
            start(a + 2, mbuf0, cbuf0, sem0)

        wait(mbuf1, cbuf1, sem1)
        pltpu.sync_copy(mbuf1, acc0.at[dstb.at[a + 1]], add=True)
        pltpu.sync_copy(cbuf1, acc2.at[srcb.at[a + 1]], add=True)
        return carry

    lax.fori_loop(0, C128 // 2, body, 0)
    plsc.subcore_barrier()
    pltpu.sync_copy(acc0.at[pl.ds(s * RPS, RPS)], zbuf)
    pltpu.sync_copy(zbuf, p0.at[c, pl.ds(s * RPS, RPS)])
    pltpu.sync_copy(acc2.at[pl.ds(s * RPS, RPS)], zbuf)
    pltpu.sync_copy(zbuf, s2.at[c, pl.ds(s * RPS, RPS)])


@functools.lru_cache(maxsize=None)
def _sc_scatter2_kernel():
    return pl.kernel(
        _sc_scatter2_body,
        out_type=(
            jax.ShapeDtypeStruct((NC, N16, H), f32),
            jax.ShapeDtypeStruct((NC, N16, H), f32),
        ),
        mesh=_mesh(),
        compiler_params=pltpu.CompilerParams(needs_layout_passes=False, use_tc_tiling_on_sc=False),
        scratch_types=[
            pltpu.VMEM_SHARED((N16, H), f32),
            pltpu.VMEM_SHARED((N16, H), f32),
            pltpu.VMEM((C128, 128), i32),
            pltpu.VMEM((C128, 128), i32),
            pltpu.VMEM((128, H), f32),
            pltpu.VMEM((128, H), f32),
            pltpu.VMEM((128, H), f32),
            pltpu.VMEM((128, H), f32),
            pltpu.VMEM((RPS, H), f32),
            pltpu.SemaphoreType.DMA,
            pltpu.SemaphoreType.DMA,
        ],
    )


def _sc_gather_mul_scatter_body(y1, callr, srcr, dstr, p1, acc, srcb,
                                dstb, rows0, cbuf0, rows1, cbuf1, zbuf,
                                sem0, sem1):
    c = lax.axis_index("c")
    s = lax.axis_index("s")
    wid = c * NS + s
    z16 = jnp.zeros((16,), f32)
    nrow = pl.multiple_of(s * RPS, 8)

    def zbody(j, carry):
        zbuf[j, pl.ds(0, 16)] = z16
        zbuf[j, pl.ds(16, 16)] = z16
        return carry

    lax.fori_loop(0, RPS, zbody, 0)
    pltpu.sync_copy(zbuf, acc.at[pl.ds(nrow, RPS)])
    crow = pl.multiple_of(wid * C128, 8)
    pltpu.sync_copy(srcr.at[pl.ds(crow, C128)], srcb)
    pltpu.sync_copy(dstr.at[pl.ds(crow, C128)], dstb)

    def start(j, rb, cb, sem):
        base = pl.multiple_of(wid * EPW + j * 128, 128)
        pltpu.async_copy(y1.at[srcb.at[j]], rb, sem)
        pltpu.async_copy(callr.at[pl.ds(base, 128), pl.ds(H, H)], cb, sem)

    def wait_mul(rb, cb, sem):
        pltpu.make_async_copy(callr.at[pl.ds(0, 128), pl.ds(0, H)], rb,
                              sem).wait()
        pltpu.make_async_copy(callr.at[pl.ds(0, 128), pl.ds(0, H)], cb,
                              sem).wait()

        def mbody(r, carry):
            rb[r, pl.ds(0, 16)] = rb[r, pl.ds(0, 16)] * cb[r, pl.ds(0, 16)]
            rb[r, pl.ds(16, 16)] = rb[r, pl.ds(16, 16)] * cb[r, pl.ds(16, 16)]
            return carry

        lax.fori_loop(0, 128, mbody, 0)

    plsc.subcore_barrier()
    start(0, rows0, cbuf0, sem0)

    def body(jo, carry):
        a = 2 * jo
        start(a + 1, rows1, cbuf1, sem1)
        wait_mul(rows0, cbuf0, sem0)
        pltpu.sync_copy(rows0, acc.at[dstb.at[a]], add=True)

        @pl.when(jo < C128 // 2 - 1)
        def _():
            start(a + 2, rows0, cbuf0, sem0)

        wait_mul(rows1, cbuf1, sem1)
        pltpu.sync_copy(rows1, acc.at[dstb.at[a + 1]], add=True)
        return carry

    lax.fori_loop(0, C128 // 2, body, 0)
    plsc.subcore_barrier()
    pltpu.sync_copy(acc.at[pl.ds(s * RPS, RPS)], zbuf)
    pltpu.sync_copy(zbuf, p1.at[c, pl.ds(s * RPS, RPS)])


@functools.lru_cache(maxsize=None)
def _sc_gather_mul_scatter_kernel():
    return pl.kernel(
        _sc_gather_mul_scatter_body,
        out_type=jax.ShapeDtypeStruct((NC, N16, H), f32),
        mesh=_mesh(),
        compiler_params=pltpu.CompilerParams(needs_layout_passes=False, use_tc_tiling_on_sc=False),
        scratch_types=[
            pltpu.VMEM_SHARED((N16, H), f32),
            pltpu.VMEM((C128, 128), i32),
            pltpu.VMEM((C128, 128), i32),
            pltpu.VMEM((128, H), f32),
            pltpu.VMEM((128, H), f32),
            pltpu.VMEM((128, H), f32),
            pltpu.VMEM((128, H), f32),
            pltpu.VMEM((RPS, H), f32),
            pltpu.SemaphoreType.DMA,
            pltpu.SemaphoreType.DMA,
        ],
    )


def _tc_edge_body(ev_ref, wf1_ref, bf1_ref, wf2_ref, wsh_ref, ws0_ref,
                  call_ref):
    x = ev_ref[0:1, :]
    y = ev_ref[1:2, :]
    z = ev_ref[2:3, :]
    d = jnp.sqrt(x * x + y * y + z * z)
    rinv = 1.0 / (d + 1e-9)
    nx = x * rinv
    ny = y * rinv
    nz = z * rinv
    s3 = np.sqrt(3.0).astype(np.float32)
    s15 = np.sqrt(15.0).astype(np.float32)
    s5 = np.sqrt(5.0).astype(np.float32)
    u = jnp.clip(d * (1.0 / MAXR), 0.0, 1.0)
    xx = np.float32(0.5 * np.pi) * u
    x2 = xx * xx
    emb = 1.0 + x2 * (np.float32(-1 / 2) + x2 * (np.float32(1 / 24) + x2 * (
        np.float32(-1 / 720) + x2 * (np.float32(1 / 40320)
                                     + x2 * np.float32(-1 / 3628800)))))
    geomT = jnp.concatenate([
        jnp.ones_like(nx), s3 * nx, s3 * ny, s3 * nz,
        s15 * nx * ny, s15 * ny * nz, 0.5 * s5 * (3.0 * nz * nz - 1.0),
        s15 * nx * nz, 0.5 * s15 * (nx * nx - ny * ny),
        emb], axis=0)
    geom = geomT.T
    sh = geom[:, 0:SH]
    embc = geom[:, SH:SH + 1]
    hid = jnp.maximum(embc * wf1_ref[...] + bf1_ref[...], 0.0)
    w_all = jnp.dot(hid, wf2_ref[...], preferred_element_type=f32)
    csh_all = jnp.dot(sh, wsh_ref[...], preferred_element_type=f32)
    call_ref[:, pl.ds(0, 3 * H)] = csh_all * w_all


def _tc_edge_feats(ev, wf1, bf1, wf2, wsh, ws0):
    full = lambda shape: pl.BlockSpec(shape, lambda b: (0,) * len(shape))
    return pl.pallas_call(
        _tc_edge_body,
        grid=(NB_E,),
        in_specs=[
            pl.BlockSpec((3, TCB), lambda b: (0, b)),
            full((1, 3 * FC)),
            full((1, 3 * FC)),
            full((3 * FC, 3 * H)),
            full((SH, 3 * H)),
            full((1, H)),
        ],
        out_specs=pl.BlockSpec((TCB, 128), lambda b: (b, 0)),
        out_shape=jax.ShapeDtypeStruct((EPAD, 128), f32),
    )(ev, wf1, bf1, wf2, wsh, ws0)


def _tc_layer1_body(p0_ref, na_ref, wo0a_ref, wo0b_ref, wo0c_ref, ws1_ref,
                    x1_ref, y1_ref):
    agg = p0_ref[0] + p0_ref[1]
    x1 = jnp.tanh(jnp.dot(agg, wo0a_ref[...]) + wo0b_ref[...]
                  + na_ref[...] * wo0c_ref[...])
    x1_ref[...] = x1
    y1_ref[...] = jnp.dot(x1, ws1_ref[...])


def _tc_layer1(p0, na, wo0a, wo0b, wo0c, ws1):
    full = lambda shape: pl.BlockSpec(shape, lambda b: (0,) * len(shape))
    return pl.pallas_call(
        _tc_layer1_body,
        grid=(NB_N,),
        in_specs=[
            pl.BlockSpec((NC, TCN, H), lambda b: (0, b, 0)),
            pl.BlockSpec((TCN, 1), lambda b: (b, 0)),
            full((H, 64)),
            full((1, 64)),
            full((1, 64)),
            full((64, H)),
        ],
        out_specs=[
            pl.BlockSpec((TCN, 64), lambda b: (b, 0)),
            pl.BlockSpec((TCN, H), lambda b: (b, 0)),
        ],
        out_shape=[
            jax.ShapeDtypeStruct((N16, 64), f32),
            jax.ShapeDtypeStruct((N16, H), f32),
        ],
    )(p0, na, wo0a, wo0b, wo0c, ws1)


def _tc_final_body(p1_ref, s2_ref, x1_ref, na_ref, wo1a_ref, wo1b_ref,
                   wo1c_ref, ws2_ref, wo2a_ref, wo2b_ref, wo2c_ref,
                   t_ref, sx2_ref, sna_ref, out_ref):
    b = pl.program_id(0)
    agg1 = p1_ref[0] + p1_ref[1]
    x2 = jnp.tanh(jnp.dot(agg1, wo1a_ref[...])
                  + jnp.dot(x1_ref[...], wo1b_ref[...])
                  + na_ref[...] * wo1c_ref[...])
    rows = lax.broadcasted_iota(i32, (TCN, 1), 0) + b * TCN
    mask = rows < N
    x2m = jnp.where(mask, x2, 0.0)
    sb = jnp.where(mask, s2_ref[0] + s2_ref[1], 0.0)
    tb = lax.dot_general(x2m, sb, (((0,), (0,)), ((), ())))
    sx2b = jnp.sum(x2m, axis=0, keepdims=True)
    snab = jnp.sum(jnp.where(mask, na_ref[...], 0.0), axis=0,
                   keepdims=True)

    @pl.when(b == 0)
    def _():
        t_ref[...] = tb
        sx2_ref[...] = sx2b
        sna_ref[...] = snab

    @pl.when(b > 0)
    def _():
        t_ref[...] = t_ref[...] + tb
        sx2_ref[...] = sx2_ref[...] + sx2b
        sna_ref[...] = sna_ref[...] + snab

    @pl.when(b == NB_N - 1)
    def _():
        sumagg2 = jnp.sum(ws2_ref[...] * t_ref[...], axis=0,
                          keepdims=True)
        out_ref[...] = (jnp.dot(sumagg2, wo2a_ref[...])
                        + jnp.dot(sx2_ref[...], wo2b_ref[...])
                        + sna_ref[...] * wo2c_ref[...]) * np.float32(
                            1.0 / np.sqrt(float(N)))


def _tc_final(p1, s2, x1, na, wo1a, wo1b, wo1c, ws2, wo2a, wo2b, wo2c):
    full = lambda shape: pl.BlockSpec(shape, lambda b: (0,) * len(shape))
    outs = pl.pallas_call(
        _tc_final_body,
        grid=(NB_N,),
        in_specs=[
            pl.BlockSpec((NC, TCN, H), lambda b: (0, b, 0)),
            pl.BlockSpec((NC, TCN, H), lambda b: (0, b, 0)),
            pl.BlockSpec((TCN, 64), lambda b: (b, 0)),
            pl.BlockSpec((TCN, 1), lambda b: (b, 0)),
            full((H, 64)),
            full((64, 64)),
            full((1, 64)),
            full((64, H)),
            full((H, 1)),
            full((64, 1)),
            full((1, 1)),
        ],
        out_specs=[full((64, H)), full((1, 64)), full((1, 1)), full((1, 1))],
        out_shape=[
            jax.ShapeDtypeStruct((64, H), f32),
            jax.ShapeDtypeStruct((1, 64), f32),
            jax.ShapeDtypeStruct((1, 1), f32),
            jax.ShapeDtypeStruct((1, 1), f32),
        ],
    )(p1, s2, x1, na, wo1a, wo1b, wo1c, ws2, wo2a, wo2b, wo2c)
    return outs[3]


def kernel(pos, node_attr, edge_index, Wf1_0, bf1_0, Wf2_0, Ws_0, Wsh_0, Wo_0,
           Wf1_1, bf1_1, Wf2_1, Ws_1, Wsh_1, Wo_1, Wf1_2, bf1_2, Wf2_2, Ws_2,
           Wsh_2, Wo_2):
    pos = pos.astype(f32)
    node_attr = node_attr.astype(f32)
    src = edge_index[0].astype(i32)
    dst = edge_index[1].astype(i32)
    padidx = jnp.full((EPAD - E,), N, i32)
    src_p = jnp.concatenate([src, padidx])
    dst_p = jnp.concatenate([dst, padidx])
    pos_p = jnp.concatenate([pos, jnp.zeros((NPAD, 3), f32)], axis=0)
    na_p = jnp.concatenate([node_attr, jnp.zeros((NPAD, 1), f32)], axis=0)

    wf1 = jnp.concatenate([Wf1_0, Wf1_1, Wf1_2], axis=1)
    bf1 = jnp.concatenate([bf1_0, bf1_1, bf1_2]).reshape(1, 3 * FC)
    z = jnp.zeros((FC, H), f32)
    wf2 = jnp.concatenate([
        jnp.concatenate([Wf2_0, z, z], axis=1),
        jnp.concatenate([z, Wf2_1, z], axis=1),
        jnp.concatenate([z, z, Wf2_2], axis=1)], axis=0)
    sc = np.float32(INV_SQRT_NEIGH)
    wsh = jnp.concatenate([Wsh_0 * (Ws_0[0:1] * sc), Wsh_1 * sc,
                           Wsh_2 * sc], axis=1)
    ws0 = Ws_0[0:1]

    ev = _sc_edge_vec_kernel()(pos_p[:, 0], pos_p[:, 1], pos_p[:, 2],
                               src_p.reshape(NW, EPW), dst_p.reshape(NW, EPW))
    call = _tc_edge_feats(ev, wf1, bf1, wf2, wsh, ws0)

    src_r = src_p.reshape(NW * C128, 128)
    dst_r = dst_p.reshape(NW * C128, 128)
    p0, s2 = _sc_scatter2_kernel()(call, dst_r, src_r)

    x1, y1 = _tc_layer1(p0, na_p, Wo_0[:H], Wo_0[H:H + 1], Wo_0[H + 18:H + 19],
                        Ws_1)

    p1 = _sc_gather_mul_scatter_kernel()(y1, call, src_r, dst_r)

    return _tc_final(p1, s2, x1, na_p, Wo_1[:H], Wo_1[H:H + 64],
                     Wo_1[H + 64:H + 65], Ws_2, Wo_2[:H], Wo_2[H:H + 64],
                     Wo_2[H + 64:H + 65])
